# Initial kernel scaffold; baseline (speedup 1.0000x reference)
import jax
import jax.numpy as jnp
from jax.experimental import pallas as pl


def kernel(z_i, z_j):
    raise NotImplementedError("write your pallas kernel here")


# speedup vs baseline: 1.3977x; 1.3977x over previous
import jax
import jax.numpy as jnp
from jax.experimental import pallas as pl
from jax.experimental.pallas import tpu as pltpu

_TEMPERATURE = 0.5
_EPS = 1e-8
_N = 8192
_HALF = 4096
_D = 256
_BR = 256
_CHUNK = 1024
_NORM_BLK = 512


def _norm_kernel(z_ref, zn_ref):
    z = z_ref[...]
    nrm = jnp.sqrt(jnp.sum(z * z, axis=1, keepdims=True))
    zn_ref[...] = (z / jnp.maximum(nrm, _EPS)).astype(jnp.bfloat16)


def _loss_kernel(rows_ref, zn_ref, out_ref):
    i = pl.program_id(0)
    r0 = i * _BR
    rows = rows_ref[...]
    row_ids = r0 + jax.lax.broadcasted_iota(jnp.int32, (_BR, _CHUNK), 0)
    pos_col = jnp.where(row_ids < _HALF, row_ids + _HALF, row_ids - _HALF)
    ssum = jnp.zeros((_BR, 1), jnp.float32)
    psum = jnp.zeros((_BR, 1), jnp.float32)
    for c in range(_N // _CHUNK):
        c0 = c * _CHUNK
        chunk = zn_ref[c0:c0 + _CHUNK, :]
        s = jax.lax.dot_general(
            rows, chunk, (((1,), (1,)), ((), ())),
            preferred_element_type=jnp.float32)
        col_ids = c0 + jax.lax.broadcasted_iota(jnp.int32, (_BR, _CHUNK), 1)
        e = jnp.where(row_ids == col_ids, 0.0,
                      jnp.exp(2.0 * s - 2.0))
        ssum = ssum + jnp.sum(e, axis=1, keepdims=True)
        psum = psum + jnp.sum(jnp.where(col_ids == pos_col, s, 0.0),
                              axis=1, keepdims=True)
    out_ref[...] = 2.0 + jnp.log(ssum) - 2.0 * psum


def kernel(z_i, z_j):
    z = jnp.concatenate([z_i, z_j], axis=0)
    zn = pl.pallas_call(
        _norm_kernel,
        out_shape=jax.ShapeDtypeStruct((_N, _D), jnp.bfloat16),
        grid=(_N // _NORM_BLK,),
        in_specs=[pl.BlockSpec((_NORM_BLK, _D), lambda i: (i, 0))],
        out_specs=pl.BlockSpec((_NORM_BLK, _D), lambda i: (i, 0)),
        compiler_params=pltpu.CompilerParams(
            dimension_semantics=("parallel",)),
        name="ntxent_norm",
    )(z)
    per_row = pl.pallas_call(
        _loss_kernel,
        out_shape=jax.ShapeDtypeStruct((_N, 1), jnp.float32),
        grid=(_N // _BR,),
        in_specs=[
            pl.BlockSpec((_BR, _D), lambda i: (i, 0)),
            pl.BlockSpec((_N, _D), lambda i: (0, 0)),
        ],
        out_specs=pl.BlockSpec((_BR, 1), lambda i: (i, 0)),
        compiler_params=pltpu.CompilerParams(
            dimension_semantics=("parallel",)),
        name="ntxent_loss",
    )(zn, zn)
    return jnp.mean(per_row)


# speedup vs baseline: 2.4617x; 1.7613x over previous
import jax
import jax.numpy as jnp
from jax.experimental import pallas as pl
from jax.experimental.pallas import tpu as pltpu

_EPS = 1e-8
_N = 8192
_D = 256
_BR = 256
_SHIFT = 4096 // _BR
_CHUNK = 1024
_NORM_BLK = 512
_TWO_LOG2E = 2.8853900817779268


def _norm_kernel(z_ref, zn_ref):
    z = z_ref[...]
    nrm = jnp.sqrt(jnp.sum(z * z, axis=1, keepdims=True))
    zn_ref[...] = (z / jnp.maximum(nrm, _EPS)).astype(jnp.bfloat16)


def _loss_kernel(rows_ref, pair_ref, zn_ref, out_ref):
    rows = rows_ref[...]
    rows_f = rows.astype(jnp.float32)
    pos = jnp.sum(rows_f * pair_ref[...].astype(jnp.float32),
                  axis=1, keepdims=True)
    self_sim = jnp.sum(rows_f * rows_f, axis=1, keepdims=True)
    ssum = jnp.zeros((_BR, 1), jnp.float32)
    for c in range(_N // _CHUNK):
        chunk = zn_ref[c * _CHUNK:(c + 1) * _CHUNK, :]
        s = jax.lax.dot_general(
            rows, chunk, (((1,), (1,)), ((), ())),
            preferred_element_type=jnp.float32)
        e = jnp.exp2(s * _TWO_LOG2E - _TWO_LOG2E)
        ssum = ssum + jnp.sum(e, axis=1, keepdims=True)
    ssum = ssum - jnp.exp2(self_sim * _TWO_LOG2E - _TWO_LOG2E)
    out_ref[...] = 2.0 + jnp.log(ssum) - 2.0 * pos


def kernel(z_i, z_j):
    z = jnp.concatenate([z_i, z_j], axis=0)
    zn = pl.pallas_call(
        _norm_kernel,
        out_shape=jax.ShapeDtypeStruct((_N, _D), jnp.bfloat16),
        grid=(_N // _NORM_BLK,),
        in_specs=[pl.BlockSpec((_NORM_BLK, _D), lambda i: (i, 0))],
        out_specs=pl.BlockSpec((_NORM_BLK, _D), lambda i: (i, 0)),
        compiler_params=pltpu.CompilerParams(
            dimension_semantics=("parallel",)),
        name="ntxent_norm",
    )(z)
    nb = _N // _BR
    per_row = pl.pallas_call(
        _loss_kernel,
        out_shape=jax.ShapeDtypeStruct((_N, 1), jnp.float32),
        grid=(nb,),
        in_specs=[
            pl.BlockSpec((_BR, _D), lambda i: (i, 0)),
            pl.BlockSpec((_BR, _D), lambda i: ((i + _SHIFT) % nb, 0)),
            pl.BlockSpec((_N, _D), lambda i: (0, 0)),
        ],
        out_specs=pl.BlockSpec((_BR, 1), lambda i: (i, 0)),
        compiler_params=pltpu.CompilerParams(
            dimension_semantics=("parallel",)),
        name="ntxent_loss",
    )(zn, zn, zn)
    return jnp.mean(per_row)


# speedup vs baseline: 2.9944x; 1.2164x over previous
import jax
import jax.numpy as jnp
from jax.experimental import pallas as pl
from jax.experimental.pallas import tpu as pltpu

_EPS = 1e-8
_HALF = 4096
_N = 8192
_D = 256
_BR = 256
_NBH = _HALF // _BR
_CHUNK = 1024
_CPH = _HALF // _CHUNK
_NORM_BLK = 512
_C = 2.8853900817779268
_SQRT_C = 1.6986436287041668


def _norm_kernel(zi_ref, zj_ref, zn_ref):
    for h, ref in enumerate((zi_ref, zj_ref)):
        z = ref[...]
        nrm = jnp.sqrt(jnp.sum(z * z, axis=1, keepdims=True))
        scl = _SQRT_C / jnp.maximum(nrm, _EPS)
        zn_ref[h] = (z * scl).astype(jnp.bfloat16)


def _loss_kernel(rows_ref, pair_ref, zn_ref, out_ref):
    rows = rows_ref[0]
    rows_f = rows.astype(jnp.float32)
    pos_c = jnp.sum(rows_f * pair_ref[0].astype(jnp.float32),
                    axis=1, keepdims=True)
    self_c = jnp.sum(rows_f * rows_f, axis=1, keepdims=True)
    ssum = jnp.zeros((_BR, 1), jnp.float32)
    for c in range(_N // _CHUNK):
        chunk = zn_ref[c // _CPH,
                       (c % _CPH) * _CHUNK:(c % _CPH + 1) * _CHUNK, :]
        s = jax.lax.dot_general(
            rows, chunk, (((1,), (1,)), ((), ())),
            preferred_element_type=jnp.float32)
        ssum = ssum + jnp.sum(jnp.exp2(s), axis=1, keepdims=True)
    out_ref[...] = jnp.log(ssum - jnp.exp2(self_c)) - (2.0 / _C) * pos_c


def kernel(z_i, z_j):
    zn = pl.pallas_call(
        _norm_kernel,
        out_shape=jax.ShapeDtypeStruct((2, _HALF, _D), jnp.bfloat16),
        grid=(_HALF // _NORM_BLK,),
        in_specs=[
            pl.BlockSpec((_NORM_BLK, _D), lambda i: (i, 0)),
            pl.BlockSpec((_NORM_BLK, _D), lambda i: (i, 0)),
        ],
        out_specs=pl.BlockSpec((2, _NORM_BLK, _D), lambda i: (0, i, 0)),
        compiler_params=pltpu.CompilerParams(
            dimension_semantics=("arbitrary",)),
        name="ntxent_norm",
    )(z_i, z_j)
    per_row = pl.pallas_call(
        _loss_kernel,
        out_shape=jax.ShapeDtypeStruct((_N, 1), jnp.float32),
        grid=(2, _NBH),
        in_specs=[
            pl.BlockSpec((1, _BR, _D), lambda h, j: (h, j, 0)),
            pl.BlockSpec((1, _BR, _D), lambda h, j: (1 - h, j, 0)),
            pl.BlockSpec((2, _HALF, _D), lambda h, j: (0, 0, 0)),
        ],
        out_specs=pl.BlockSpec((_BR, 1), lambda h, j: (h * _NBH + j, 0)),
        compiler_params=pltpu.CompilerParams(
            dimension_semantics=("arbitrary", "arbitrary")),
        name="ntxent_loss",
    )(zn, zn, zn)
    return jnp.mean(per_row)


# speedup vs baseline: 3.3598x; 1.1220x over previous
import jax
import jax.numpy as jnp
from jax.experimental import pallas as pl
from jax.experimental.pallas import tpu as pltpu

_EPS = 1e-8
_HALF = 4096
_N = 8192
_D = 256
_BR = 256
_NBH = _HALF // _BR
_CHUNK = 1024
_CPH = _HALF // _CHUNK
_NORM_BLK = 512
_C = 2.8853900817779268
_SQRT_C = 1.6986436287041668


def _ntxent_kernel(zi_ref, zj_ref, out_ref, zn_ref):
    i = pl.program_id(0)

    @pl.when(i == 0)
    def _normalize():
        for h, ref in enumerate((zi_ref, zj_ref)):
            for k in range(_HALF // _NORM_BLK):
                z = ref[k * _NORM_BLK:(k + 1) * _NORM_BLK, :]
                nrm = jnp.sqrt(jnp.sum(z * z, axis=1, keepdims=True))
                scl = _SQRT_C / jnp.maximum(nrm, _EPS)
                zn_ref[h, k * _NORM_BLK:(k + 1) * _NORM_BLK, :] = (
                    (z * scl).astype(jnp.bfloat16))

    h = i // _NBH
    j = i % _NBH
    rows = zn_ref[h, pl.ds(j * _BR, _BR), :]
    pair = zn_ref[1 - h, pl.ds(j * _BR, _BR), :]
    rows_f = rows.astype(jnp.float32)
    pos_c = jnp.sum(rows_f * pair.astype(jnp.float32),
                    axis=1, keepdims=True)
    self_c = jnp.sum(rows_f * rows_f, axis=1, keepdims=True)
    ssum = jnp.zeros((_BR, 1), jnp.float32)
    for c in range(_N // _CHUNK):
        chunk = zn_ref[c // _CPH,
                       (c % _CPH) * _CHUNK:(c % _CPH + 1) * _CHUNK, :]
        s = jax.lax.dot_general(
            rows, chunk, (((1,), (1,)), ((), ())),
            preferred_element_type=jnp.float32)
        ssum = ssum + jnp.sum(jnp.exp2(s), axis=1, keepdims=True)
    out_ref[...] = jnp.log(ssum - jnp.exp2(self_c)) - (2.0 / _C) * pos_c


def kernel(z_i, z_j):
    per_row = pl.pallas_call(
        _ntxent_kernel,
        out_shape=jax.ShapeDtypeStruct((_N, 1), jnp.float32),
        grid=(_N // _BR,),
        in_specs=[
            pl.BlockSpec((_HALF, _D), lambda i: (0, 0)),
            pl.BlockSpec((_HALF, _D), lambda i: (0, 0)),
        ],
        out_specs=pl.BlockSpec((_BR, 1), lambda i: (i, 0)),
        scratch_shapes=[pltpu.VMEM((2, _HALF, _D), jnp.bfloat16)],
        compiler_params=pltpu.CompilerParams(
            dimension_semantics=("arbitrary",),
            vmem_limit_bytes=50 * 1024 * 1024),
        name="ntxent_loss",
    )(z_i, z_j)
    return jnp.mean(per_row)
